# baseline (speedup 1.0000x reference)
import functools

import jax
import jax.numpy as jnp
from jax import lax
from jax.experimental import pallas as pl
from jax.experimental.pallas import tpu as pltpu
from jax.experimental.pallas import tpu_sc as plsc

N_TOK = 16384
K = 1024
D = 256
ROWS = 1024
GRID = N_TOK // ROWS
C = 128
NCH = K // C

NW = 32
B_PER_W = N_TOK // NW
CH = 4
ROWS_CH = B_PER_W // CH


def _dist_argmin_body(z_ref, e_ref, idx_ref, minsum_ref,
                      zsq_ref, runmin_ref, runidx_ref):
    i = pl.program_id(0)
    j = pl.program_id(1)
    z = z_ref[...]
    ec = e_ref[...]

    @pl.when(j == 0)
    def _():
        zsq_ref[...] = jnp.sum(z * z, axis=1)

    esq = jnp.sum(ec * ec, axis=1)
    prod = lax.dot_general(ec, z, (((1,), (1,)), ((), ())),
                           preferred_element_type=jnp.float32)
    dist = (zsq_ref[...][None, :] + esq[:, None]) - 2.0 * prod
    bmin = jnp.min(dist, axis=0)
    iota = lax.broadcasted_iota(jnp.int32, dist.shape, 0).astype(jnp.float32)
    bidx = jnp.min(jnp.where(dist == bmin[None, :], iota, jnp.float32(C)),
                   axis=0) + jnp.float32(j * C)

    @pl.when(j == 0)
    def _():
        runmin_ref[...] = bmin
        runidx_ref[...] = bidx

    @pl.when(j > 0)
    def _():
        better = bmin < runmin_ref[...]
        runidx_ref[...] = jnp.where(better, bidx, runidx_ref[...])
        runmin_ref[...] = jnp.minimum(bmin, runmin_ref[...])

    @pl.when(jnp.logical_and(i == 0, j == 0))
    def _():
        minsum_ref[0, 0] = 0.0

    @pl.when(j == NCH - 1)
    def _():
        idx_ref[0, 0, :] = runidx_ref[...].astype(jnp.int32)
        minsum_ref[0, 0] += jnp.sum(runmin_ref[...])


def _dist_argmin(z_flat, embedding):
    return pl.pallas_call(
        _dist_argmin_body,
        grid=(GRID, NCH),
        in_specs=[
            pl.BlockSpec((ROWS, D), lambda i, j: (i, 0)),
            pl.BlockSpec((C, D), lambda i, j: (j, 0)),
        ],
        out_specs=[
            pl.BlockSpec((1, 1, ROWS), lambda i, j: (i, 0, 0)),
            pl.BlockSpec(memory_space=pltpu.SMEM),
        ],
        out_shape=[
            jax.ShapeDtypeStruct((GRID, 1, ROWS), jnp.int32),
            jax.ShapeDtypeStruct((1, 1), jnp.float32),
        ],
        scratch_shapes=[
            pltpu.VMEM((ROWS,), jnp.float32),
            pltpu.VMEM((ROWS,), jnp.float32),
            pltpu.VMEM((ROWS,), jnp.float32),
        ],
    )(z_flat, embedding)


def _sc_gather(embedding, idx_grouped):
    mesh = plsc.VectorSubcoreMesh(core_axis_name="c", subcore_axis_name="s")

    @functools.partial(
        pl.kernel,
        mesh=mesh,
        out_type=jax.ShapeDtypeStruct((N_TOK, D), jnp.float32),
        scratch_types=[
            pltpu.VMEM((CH, ROWS_CH), jnp.int32),
            pltpu.VMEM((ROWS_CH, D), jnp.float32),
            pltpu.SemaphoreType.DMA,
        ],
    )
    def gather_k(table_hbm, idx_hbm, out_hbm, idx_v, rows_v, sem):
        wid = lax.axis_index("s") * 2 + lax.axis_index("c")
        base = wid * B_PER_W
        pltpu.sync_copy(idx_hbm.at[wid], idx_v)
        for c in range(CH):
            pltpu.async_copy(table_hbm.at[idx_v.at[c]], rows_v, sem).wait()
            pltpu.sync_copy(rows_v, out_hbm.at[pl.ds(base + c * ROWS_CH, ROWS_CH)])

    return gather_k(embedding, idx_grouped)


def kernel(z_flat, embedding):
    idx3, minsum = _dist_argmin(z_flat, embedding)
    indices = idx3.reshape(N_TOK)
    loss = minsum[0, 0] * (1.25 / (N_TOK * D))
    quantized = _sc_gather(embedding, indices.reshape(NW, CH, ROWS_CH))
    return (loss, quantized, indices)


# speedup vs baseline: 1.5270x; 1.5270x over previous
import functools

import jax
import jax.numpy as jnp
from jax import lax
from jax.experimental import pallas as pl
from jax.experimental.pallas import tpu as pltpu
from jax.experimental.pallas import tpu_sc as plsc

N_TOK = 16384
K = 1024
D = 256
ROWS = 1024
GRID = N_TOK // ROWS
C = 128
NCH = K // C

NW = 32
B_PER_W = N_TOK // NW
CH = 4
ROWS_CH = B_PER_W // CH


def _dist_argmin_body(z_ref, e_ref, idx_ref, minsum_ref, esq_ref):
    i = pl.program_id(0)
    z = z_ref[...]

    @pl.when(i == 0)
    def _():
        e = e_ref[...]
        esq_ref[...] = jnp.sum(e * e, axis=1, keepdims=True)

    zsq_col = jnp.sum(z * z, axis=1, keepdims=True)
    zsq = zsq_col.T
    z2 = z * (-2.0)
    subiota = lax.broadcasted_iota(jnp.int32, (8, ROWS), 0).astype(jnp.float32)

    def chunk(j, carry):
        rmin, ridx = carry
        ec = e_ref[pl.ds(j * C, C), :]
        prod2 = lax.dot_general(ec, z2, (((1,), (1,)), ((), ())),
                                preferred_element_type=jnp.float32)
        esq_c = esq_ref[pl.ds(j * C, C), :]
        dist = (zsq + esq_c) + prod2
        kbase = (j * (C // 8)).astype(jnp.float32)
        for k in range(C // 8):
            blk = lax.slice(dist, (8 * k, 0), (8 * k + 8, ROWS))
            better = blk < rmin
            rmin = jnp.where(better, blk, rmin)
            ridx = jnp.where(better, kbase + jnp.float32(k), ridx)
        return rmin, ridx

    rmin = jnp.full((8, ROWS), 3.0e38, jnp.float32)
    ridx = jnp.zeros((8, ROWS), jnp.float32)
    rmin, ridx = lax.fori_loop(0, NCH, chunk, (rmin, ridx))

    idxf = ridx * 8.0 + subiota
    vals, idxs = rmin, idxf
    for half in (4, 2, 1):
        vt, vb = vals[:half, :], vals[half:2 * half, :]
        it, ib = idxs[:half, :], idxs[half:2 * half, :]
        take_b = jnp.logical_or(vb < vt,
                                jnp.logical_and(vb == vt, ib < it))
        vals = jnp.where(take_b, vb, vt)
        idxs = jnp.where(take_b, ib, it)

    idx_ref[0, 0, :] = idxs[0, :].astype(jnp.int32)

    @pl.when(i == 0)
    def _():
        minsum_ref[0, 0] = 0.0

    minsum_ref[0, 0] += jnp.sum(vals[0, :])


def _dist_argmin(z_flat, embedding):
    return pl.pallas_call(
        _dist_argmin_body,
        grid=(GRID,),
        in_specs=[
            pl.BlockSpec((ROWS, D), lambda i: (i, 0)),
            pl.BlockSpec((K, D), lambda i: (0, 0)),
        ],
        out_specs=[
            pl.BlockSpec((1, 1, ROWS), lambda i: (i, 0, 0)),
            pl.BlockSpec(memory_space=pltpu.SMEM),
        ],
        out_shape=[
            jax.ShapeDtypeStruct((GRID, 1, ROWS), jnp.int32),
            jax.ShapeDtypeStruct((1, 1), jnp.float32),
        ],
        scratch_shapes=[
            pltpu.VMEM((K, 1), jnp.float32),
        ],
    )(z_flat, embedding)


def _sc_gather(embedding, idx_grouped):
    mesh = plsc.VectorSubcoreMesh(core_axis_name="c", subcore_axis_name="s")

    @functools.partial(
        pl.kernel,
        mesh=mesh,
        out_type=jax.ShapeDtypeStruct((N_TOK, D), jnp.float32),
        scratch_types=[
            pltpu.VMEM((CH, ROWS_CH), jnp.int32),
            pltpu.VMEM((ROWS_CH, D), jnp.float32),
            pltpu.SemaphoreType.DMA,
        ],
    )
    def gather_k(table_hbm, idx_hbm, out_hbm, idx_v, rows_v, sem):
        wid = lax.axis_index("s") * 2 + lax.axis_index("c")
        base = wid * B_PER_W
        pltpu.sync_copy(idx_hbm.at[wid], idx_v)
        for c in range(CH):
            pltpu.async_copy(table_hbm.at[idx_v.at[c]], rows_v, sem).wait()
            pltpu.sync_copy(rows_v, out_hbm.at[pl.ds(base + c * ROWS_CH, ROWS_CH)])

    return gather_k(embedding, idx_grouped)


def kernel(z_flat, embedding):
    idx3, minsum = _dist_argmin(z_flat, embedding)
    indices = idx3.reshape(N_TOK)
    loss = minsum[0, 0] * (1.25 / (N_TOK * D))
    quantized = _sc_gather(embedding, indices.reshape(NW, CH, ROWS_CH))
    return (loss, quantized, indices)


# speedup vs baseline: 1.7122x; 1.1213x over previous
import functools

import jax
import jax.numpy as jnp
from jax import lax
from jax.experimental import pallas as pl
from jax.experimental.pallas import tpu as pltpu
from jax.experimental.pallas import tpu_sc as plsc

N_TOK = 16384
K = 1024
D = 256
ROWS = 1024
GRID = N_TOK // ROWS
C = 128
NCH = K // C

NW = 32
B_PER_W = N_TOK // NW
CH = 4
ROWS_CH = B_PER_W // CH


def _dist_argmin_body(z_ref, e_ref, idx_ref, minsum_ref, esq_ref):
    i = pl.program_id(0)
    z = z_ref[...]

    @pl.when(i == 0)
    def _():
        e = e_ref[...]
        esq_ref[...] = jnp.sum(e * e, axis=1, keepdims=True)

    zsq_col = jnp.sum(z * z, axis=1, keepdims=True)
    zsq = zsq_col.T
    z2 = z * (-2.0)
    subiota = lax.broadcasted_iota(jnp.int32, (8, ROWS), 0).astype(jnp.float32)

    def chunk(j, carry):
        rmin, ridx = carry
        del carry
        ec = e_ref[pl.ds(j * C, C), :]
        prod2 = lax.dot_general(ec, z2, (((1,), (1,)), ((), ())),
                                preferred_element_type=jnp.float32)
        esq_c = esq_ref[pl.ds(j * C, C), :]
        dist = (zsq + esq_c) + prod2
        kbase = (j * (C // 8)).astype(jnp.float32)
        for k in range(C // 8):
            blk = lax.slice(dist, (8 * k, 0), (8 * k + 8, ROWS))
            better = blk < rmin
            rmin = jnp.where(better, blk, rmin)
            ridx = jnp.where(better, kbase + jnp.float32(k), ridx)
        return rmin, ridx

    def pair(p, carry):
        carry = chunk(2 * p, carry)
        carry = chunk(2 * p + 1, carry)
        return carry

    rmin = jnp.full((8, ROWS), 3.0e38, jnp.float32)
    ridx = jnp.zeros((8, ROWS), jnp.float32)
    rmin, ridx = lax.fori_loop(0, NCH // 2, pair, (rmin, ridx))

    idxf = ridx * 8.0 + subiota
    vals, idxs = rmin, idxf
    for half in (4, 2, 1):
        vt, vb = vals[:half, :], vals[half:2 * half, :]
        it, ib = idxs[:half, :], idxs[half:2 * half, :]
        take_b = jnp.logical_or(vb < vt,
                                jnp.logical_and(vb == vt, ib < it))
        vals = jnp.where(take_b, vb, vt)
        idxs = jnp.where(take_b, ib, it)

    idx_ref[0, 0, :] = idxs[0, :].astype(jnp.int32)

    @pl.when(i == 0)
    def _():
        minsum_ref[0, 0] = 0.0

    minsum_ref[0, 0] += jnp.sum(vals[0, :])


def _dist_argmin(z_flat, embedding):
    return pl.pallas_call(
        _dist_argmin_body,
        grid=(GRID,),
        in_specs=[
            pl.BlockSpec((ROWS, D), lambda i: (i, 0)),
            pl.BlockSpec((K, D), lambda i: (0, 0)),
        ],
        out_specs=[
            pl.BlockSpec((1, 1, ROWS), lambda i: (i, 0, 0)),
            pl.BlockSpec(memory_space=pltpu.SMEM),
        ],
        out_shape=[
            jax.ShapeDtypeStruct((GRID, 1, ROWS), jnp.int32),
            jax.ShapeDtypeStruct((1, 1), jnp.float32),
        ],
        scratch_shapes=[
            pltpu.VMEM((K, 1), jnp.float32),
        ],
    )(z_flat, embedding)


def _sc_gather(embedding, idx_grouped):
    mesh = plsc.VectorSubcoreMesh(core_axis_name="c", subcore_axis_name="s")

    @functools.partial(
        pl.kernel,
        mesh=mesh,
        out_type=jax.ShapeDtypeStruct((N_TOK, D), jnp.float32),
        scratch_types=[
            pltpu.VMEM((CH, ROWS_CH), jnp.int32),
            pltpu.VMEM((ROWS_CH, D), jnp.float32),
            pltpu.SemaphoreType.DMA,
        ],
    )
    def gather_k(table_hbm, idx_hbm, out_hbm, idx_v, rows_v, sem):
        wid = lax.axis_index("s") * 2 + lax.axis_index("c")
        base = wid * B_PER_W
        pltpu.sync_copy(idx_hbm.at[wid], idx_v)
        for c in range(CH):
            pltpu.async_copy(table_hbm.at[idx_v.at[c]], rows_v, sem).wait()
            pltpu.sync_copy(rows_v, out_hbm.at[pl.ds(base + c * ROWS_CH, ROWS_CH)])

    return gather_k(embedding, idx_grouped)


def kernel(z_flat, embedding):
    idx3, minsum = _dist_argmin(z_flat, embedding)
    indices = idx3.reshape(N_TOK)
    loss = minsum[0, 0] * (1.25 / (N_TOK * D))
    quantized = _sc_gather(embedding, indices.reshape(NW, CH, ROWS_CH))
    return (loss, quantized, indices)


# speedup vs baseline: 2.0213x; 1.1806x over previous
import functools

import jax
import jax.numpy as jnp
from jax import lax
from jax.experimental import pallas as pl
from jax.experimental.pallas import tpu as pltpu
from jax.experimental.pallas import tpu_sc as plsc

N_TOK = 16384
K = 1024
D = 256
ROWS = 1024
GRID = N_TOK // ROWS
C = 128
NCH = K // C

NW = 32
B_PER_W = N_TOK // NW
CH = 4
ROWS_CH = B_PER_W // CH


def _dist_argmin_body(z_ref, e_ref, idx_ref, minsum_ref, esq_ref, e2_ref):
    i = pl.program_id(0)
    z = z_ref[...]

    @pl.when(i == 0)
    def _():
        e = e_ref[...]
        esq_ref[...] = jnp.sum(e * e, axis=1, keepdims=True)
        e2_ref[...] = e * (-2.0)

    zsq_col = jnp.sum(z * z, axis=1, keepdims=True)
    zsq = zsq_col.T
    subiota = lax.broadcasted_iota(jnp.int32, (8, ROWS), 0).astype(jnp.float32)

    def chunk(j, carry):
        rmin, ridx = carry
        del carry
        ec2 = e2_ref[pl.ds(j * C, C), :]
        prod2 = lax.dot_general(ec2, z, (((1,), (1,)), ((), ())),
                                preferred_element_type=jnp.float32)
        esq_c = esq_ref[pl.ds(j * C, C), :]
        dist = (zsq + esq_c) + prod2
        kbase = (j * (C // 8)).astype(jnp.float32)
        for k in range(C // 8):
            blk = lax.slice(dist, (8 * k, 0), (8 * k + 8, ROWS))
            better = blk < rmin
            rmin = jnp.where(better, blk, rmin)
            ridx = jnp.where(better, kbase + jnp.float32(k), ridx)
        return rmin, ridx

    carry = (jnp.full((8, ROWS), 3.0e38, jnp.float32),
             jnp.zeros((8, ROWS), jnp.float32))
    for j in range(NCH):
        carry = chunk(jnp.int32(j), carry)
    rmin, ridx = carry

    idxf = ridx * 8.0 + subiota
    vals, idxs = rmin, idxf
    for half in (4, 2, 1):
        vt, vb = vals[:half, :], vals[half:2 * half, :]
        it, ib = idxs[:half, :], idxs[half:2 * half, :]
        take_b = jnp.logical_or(vb < vt,
                                jnp.logical_and(vb == vt, ib < it))
        vals = jnp.where(take_b, vb, vt)
        idxs = jnp.where(take_b, ib, it)

    idx_ref[0, 0, :] = idxs[0, :].astype(jnp.int32)

    @pl.when(i == 0)
    def _():
        minsum_ref[0, 0] = 0.0

    minsum_ref[0, 0] += jnp.sum(vals[0, :])


def _dist_argmin(z_flat, embedding):
    return pl.pallas_call(
        _dist_argmin_body,
        grid=(GRID,),
        in_specs=[
            pl.BlockSpec((ROWS, D), lambda i: (i, 0)),
            pl.BlockSpec((K, D), lambda i: (0, 0)),
        ],
        out_specs=[
            pl.BlockSpec((1, 1, ROWS), lambda i: (i, 0, 0)),
            pl.BlockSpec(memory_space=pltpu.SMEM),
        ],
        out_shape=[
            jax.ShapeDtypeStruct((GRID, 1, ROWS), jnp.int32),
            jax.ShapeDtypeStruct((1, 1), jnp.float32),
        ],
        scratch_shapes=[
            pltpu.VMEM((K, 1), jnp.float32),
            pltpu.VMEM((K, D), jnp.float32),
        ],
    )(z_flat, embedding)


def _sc_gather(embedding, idx_grouped):
    mesh = plsc.VectorSubcoreMesh(core_axis_name="c", subcore_axis_name="s")

    @functools.partial(
        pl.kernel,
        mesh=mesh,
        out_type=jax.ShapeDtypeStruct((N_TOK, D), jnp.float32),
        scratch_types=[
            pltpu.VMEM((CH, ROWS_CH), jnp.int32),
            pltpu.VMEM((ROWS_CH, D), jnp.float32),
            pltpu.SemaphoreType.DMA,
        ],
    )
    def gather_k(table_hbm, idx_hbm, out_hbm, idx_v, rows_v, sem):
        wid = lax.axis_index("s") * 2 + lax.axis_index("c")
        base = wid * B_PER_W
        pltpu.sync_copy(idx_hbm.at[wid], idx_v)
        for c in range(CH):
            pltpu.async_copy(table_hbm.at[idx_v.at[c]], rows_v, sem).wait()
            pltpu.sync_copy(rows_v, out_hbm.at[pl.ds(base + c * ROWS_CH, ROWS_CH)])

    return gather_k(embedding, idx_grouped)


def kernel(z_flat, embedding):
    idx3, minsum = _dist_argmin(z_flat, embedding)
    indices = idx3.reshape(N_TOK)
    loss = minsum[0, 0] * (1.25 / (N_TOK * D))
    quantized = _sc_gather(embedding, indices.reshape(NW, CH, ROWS_CH))
    return (loss, quantized, indices)


# speedup vs baseline: 2.1296x; 1.0536x over previous
import functools

import jax
import jax.numpy as jnp
from jax import lax
from jax.experimental import pallas as pl
from jax.experimental.pallas import tpu as pltpu
from jax.experimental.pallas import tpu_sc as plsc

N_TOK = 16384
K = 1024
D = 256
ROWS = 1024
GRID = N_TOK // ROWS
C = 128
NCH = K // C

NW = 32
B_PER_W = N_TOK // NW
CH = 4
ROWS_CH = B_PER_W // CH


def _dist_argmin_body(z_ref, e_ref, idx_ref, minsum_ref, esq_ref, e2_ref):
    i = pl.program_id(0)
    z = z_ref[...]

    @pl.when(i == 0)
    def _():
        e = e_ref[...]
        esq_ref[...] = jnp.sum(e * e, axis=1, keepdims=True)
        e2_ref[...] = e * (-2.0)

    zsq_col = jnp.sum(z * z, axis=1, keepdims=True)
    zsq = zsq_col.T
    subiota = lax.broadcasted_iota(jnp.int32, (8, ROWS), 0).astype(jnp.float32)

    def chunk(j, carry):
        rmin, ridx = carry
        del carry
        ec2 = e2_ref[pl.ds(j * C, C), :]
        prod2 = lax.dot_general(ec2, z, (((1,), (1,)), ((), ())),
                                preferred_element_type=jnp.float32)
        esq_c = esq_ref[pl.ds(j * C, C), :]
        dist = (zsq + esq_c) + prod2
        kbase = (j * (C // 8)).astype(jnp.float32)
        for k in range(C // 8):
            blk = lax.slice(dist, (8 * k, 0), (8 * k + 8, ROWS))
            better = blk < rmin
            rmin = jnp.where(better, blk, rmin)
            ridx = jnp.where(better, kbase + jnp.float32(k), ridx)
        return rmin, ridx

    carry = (jnp.full((8, ROWS), 3.0e38, jnp.float32),
             jnp.zeros((8, ROWS), jnp.float32))
    for j in range(NCH):
        carry = chunk(jnp.int32(j), carry)
    rmin, ridx = carry

    idxf = ridx * 8.0 + subiota
    vals, idxs = rmin, idxf
    for half in (4, 2, 1):
        vt, vb = vals[:half, :], vals[half:2 * half, :]
        it, ib = idxs[:half, :], idxs[half:2 * half, :]
        take_b = jnp.logical_or(vb < vt,
                                jnp.logical_and(vb == vt, ib < it))
        vals = jnp.where(take_b, vb, vt)
        idxs = jnp.where(take_b, ib, it)

    idx_ref[0, 0, :] = idxs[0, :].astype(jnp.int32)

    @pl.when(i == 0)
    def _():
        minsum_ref[0, 0] = 0.0

    minsum_ref[0, 0] += jnp.sum(vals[0, :])

    @pl.when(i == GRID - 1)
    def _():
        minsum_ref[0, 0] = minsum_ref[0, 0] * (1.25 / (N_TOK * D))


def _dist_argmin(z_flat, embedding):
    return pl.pallas_call(
        _dist_argmin_body,
        grid=(GRID,),
        in_specs=[
            pl.BlockSpec((ROWS, D), lambda i: (i, 0)),
            pl.BlockSpec((K, D), lambda i: (0, 0)),
        ],
        out_specs=[
            pl.BlockSpec((1, 1, ROWS), lambda i: (i, 0, 0)),
            pl.BlockSpec(memory_space=pltpu.SMEM),
        ],
        out_shape=[
            jax.ShapeDtypeStruct((GRID, 1, ROWS), jnp.int32),
            jax.ShapeDtypeStruct((1, 1), jnp.float32),
        ],
        scratch_shapes=[
            pltpu.VMEM((K, 1), jnp.float32),
            pltpu.VMEM((K, D), jnp.float32),
        ],
    )(z_flat, embedding)


def _sc_gather(embedding, idx_grouped):
    mesh = plsc.VectorSubcoreMesh(core_axis_name="c", subcore_axis_name="s")

    @functools.partial(
        pl.kernel,
        mesh=mesh,
        out_type=jax.ShapeDtypeStruct((N_TOK, D), jnp.float32),
        scratch_types=[
            pltpu.VMEM((CH, ROWS_CH), jnp.int32),
            pltpu.VMEM((ROWS_CH, D), jnp.float32),
            pltpu.VMEM((ROWS_CH, D), jnp.float32),
            pltpu.SemaphoreType.DMA,
            pltpu.SemaphoreType.DMA,
        ],
    )
    def gather_k(table_hbm, idx_hbm, out_hbm, idx_v, rows_a, rows_b, gsem, ssem):
        wid = lax.axis_index("s") * 2 + lax.axis_index("c")
        base = wid * B_PER_W
        bufs = (rows_a, rows_b)
        pltpu.sync_copy(idx_hbm.at[wid], idx_v)
        g = pltpu.async_copy(table_hbm.at[idx_v.at[0]], bufs[0], gsem)
        prev_s = None
        for c in range(CH):
            if prev_s is not None:
                prev_s.wait()
            if c + 1 < CH:
                g_next = pltpu.async_copy(table_hbm.at[idx_v.at[c + 1]],
                                          bufs[(c + 1) % 2], gsem)
            g.wait()
            prev_s = pltpu.async_copy(
                bufs[c % 2], out_hbm.at[pl.ds(base + c * ROWS_CH, ROWS_CH)], ssem)
            if c + 1 < CH:
                g = g_next
        prev_s.wait()

    return gather_k(embedding, idx_grouped)


def kernel(z_flat, embedding):
    idx3, minsum = _dist_argmin(z_flat, embedding)
    indices = idx3.reshape(N_TOK)
    loss = minsum[0, 0]
    quantized = _sc_gather(embedding, indices.reshape(NW, CH, ROWS_CH))
    return (loss, quantized, indices)


# speedup vs baseline: 2.1602x; 1.0143x over previous
import functools

import jax
import jax.numpy as jnp
from jax import lax
from jax.experimental import pallas as pl
from jax.experimental.pallas import tpu as pltpu
from jax.experimental.pallas import tpu_sc as plsc

N_TOK = 16384
K = 1024
D = 256
ROWS = 2048
GRID = N_TOK // ROWS
C = 128
NCH = K // C

NW = 32
B_PER_W = N_TOK // NW
CH = 4
ROWS_CH = B_PER_W // CH


def _dist_argmin_body(z_ref, e_ref, idx_ref, minsum_ref, esq_ref, e2_ref):
    i = pl.program_id(0)
    z = z_ref[...]

    @pl.when(i == 0)
    def _():
        e = e_ref[...]
        esq_ref[...] = jnp.sum(e * e, axis=1, keepdims=True)
        e2_ref[...] = e * (-2.0)

    zsq_col = jnp.sum(z * z, axis=1, keepdims=True)
    zsq = zsq_col.T
    subiota = lax.broadcasted_iota(jnp.int32, (8, ROWS), 0).astype(jnp.float32)

    def chunk(j, carry):
        rmin, ridx = carry
        del carry
        ec2 = e2_ref[pl.ds(j * C, C), :]
        prod2 = lax.dot_general(ec2, z, (((1,), (1,)), ((), ())),
                                preferred_element_type=jnp.float32)
        esq_c = esq_ref[pl.ds(j * C, C), :]
        dist = (zsq + esq_c) + prod2
        kbase = (j * (C // 8)).astype(jnp.float32)
        for k in range(C // 8):
            blk = lax.slice(dist, (8 * k, 0), (8 * k + 8, ROWS))
            better = blk < rmin
            rmin = jnp.where(better, blk, rmin)
            ridx = jnp.where(better, kbase + jnp.float32(k), ridx)
        return rmin, ridx

    carry = (jnp.full((8, ROWS), 3.0e38, jnp.float32),
             jnp.zeros((8, ROWS), jnp.float32))
    for j in range(NCH):
        carry = chunk(jnp.int32(j), carry)
    rmin, ridx = carry

    idxf = ridx * 8.0 + subiota
    vals, idxs = rmin, idxf
    for half in (4, 2, 1):
        vt, vb = vals[:half, :], vals[half:2 * half, :]
        it, ib = idxs[:half, :], idxs[half:2 * half, :]
        take_b = jnp.logical_or(vb < vt,
                                jnp.logical_and(vb == vt, ib < it))
        vals = jnp.where(take_b, vb, vt)
        idxs = jnp.where(take_b, ib, it)

    idx_ref[0, 0, :] = idxs[0, :].astype(jnp.int32)

    @pl.when(i == 0)
    def _():
        minsum_ref[0, 0] = 0.0

    minsum_ref[0, 0] += jnp.sum(vals[0, :])

    @pl.when(i == GRID - 1)
    def _():
        minsum_ref[0, 0] = minsum_ref[0, 0] * (1.25 / (N_TOK * D))


def _dist_argmin(z_flat, embedding):
    return pl.pallas_call(
        _dist_argmin_body,
        grid=(GRID,),
        in_specs=[
            pl.BlockSpec((ROWS, D), lambda i: (i, 0)),
            pl.BlockSpec((K, D), lambda i: (0, 0)),
        ],
        out_specs=[
            pl.BlockSpec((1, 1, ROWS), lambda i: (i, 0, 0)),
            pl.BlockSpec(memory_space=pltpu.SMEM),
        ],
        out_shape=[
            jax.ShapeDtypeStruct((GRID, 1, ROWS), jnp.int32),
            jax.ShapeDtypeStruct((1, 1), jnp.float32),
        ],
        scratch_shapes=[
            pltpu.VMEM((K, 1), jnp.float32),
            pltpu.VMEM((K, D), jnp.float32),
        ],
    )(z_flat, embedding)


def _sc_gather(embedding, idx_grouped):
    mesh = plsc.VectorSubcoreMesh(core_axis_name="c", subcore_axis_name="s")

    @functools.partial(
        pl.kernel,
        mesh=mesh,
        out_type=jax.ShapeDtypeStruct((N_TOK, D), jnp.float32),
        scratch_types=[
            pltpu.VMEM((CH, ROWS_CH), jnp.int32),
            pltpu.VMEM((ROWS_CH, D), jnp.float32),
            pltpu.VMEM((ROWS_CH, D), jnp.float32),
            pltpu.SemaphoreType.DMA,
            pltpu.SemaphoreType.DMA,
        ],
    )
    def gather_k(table_hbm, idx_hbm, out_hbm, idx_v, rows_a, rows_b, gsem, ssem):
        wid = lax.axis_index("s") * 2 + lax.axis_index("c")
        base = wid * B_PER_W
        bufs = (rows_a, rows_b)
        pltpu.sync_copy(idx_hbm.at[wid], idx_v)
        g = pltpu.async_copy(table_hbm.at[idx_v.at[0]], bufs[0], gsem)
        prev_s = None
        for c in range(CH):
            if prev_s is not None:
                prev_s.wait()
            if c + 1 < CH:
                g_next = pltpu.async_copy(table_hbm.at[idx_v.at[c + 1]],
                                          bufs[(c + 1) % 2], gsem)
            g.wait()
            prev_s = pltpu.async_copy(
                bufs[c % 2], out_hbm.at[pl.ds(base + c * ROWS_CH, ROWS_CH)], ssem)
            if c + 1 < CH:
                g = g_next
        prev_s.wait()

    return gather_k(embedding, idx_grouped)


def kernel(z_flat, embedding):
    idx3, minsum = _dist_argmin(z_flat, embedding)
    indices = idx3.reshape(N_TOK)
    loss = minsum[0, 0]
    quantized = _sc_gather(embedding, indices.reshape(NW, CH, ROWS_CH))
    return (loss, quantized, indices)


# speedup vs baseline: 2.4506x; 1.1344x over previous
import functools

import jax
import jax.numpy as jnp
from jax import lax
from jax.experimental import pallas as pl
from jax.experimental.pallas import tpu as pltpu
from jax.experimental.pallas import tpu_sc as plsc

N_TOK = 16384
K = 1024
D = 256
ROWS = 2048
GRID = N_TOK // ROWS
C = 512
NCH = K // C

NW = 32
B_PER_W = N_TOK // NW
CH = 4
ROWS_CH = B_PER_W // CH


def _dist_argmin_body(z_ref, e_ref, idx_ref, minsum_ref, esq_ref, e2_ref):
    i = pl.program_id(0)
    z = z_ref[...]

    @pl.when(i == 0)
    def _():
        e = e_ref[...]
        esq_ref[...] = jnp.sum(e * e, axis=1, keepdims=True)
        e2_ref[...] = e * (-2.0)

    zsq_col = jnp.sum(z * z, axis=1, keepdims=True)
    zsq = zsq_col.T
    subiota = lax.broadcasted_iota(jnp.int32, (8, ROWS), 0).astype(jnp.float32)

    def chunk(j, carry):
        rmin, ridx = carry
        del carry
        ec2 = e2_ref[pl.ds(j * C, C), :]
        prod2 = lax.dot_general(ec2, z, (((1,), (1,)), ((), ())),
                                preferred_element_type=jnp.float32)
        esq_c = esq_ref[pl.ds(j * C, C), :]
        dist = (zsq + esq_c) + prod2
        kbase = (j * (C // 8)).astype(jnp.float32)
        for k in range(C // 8):
            blk = lax.slice(dist, (8 * k, 0), (8 * k + 8, ROWS))
            better = blk < rmin
            rmin = jnp.where(better, blk, rmin)
            ridx = jnp.where(better, kbase + jnp.float32(k), ridx)
        return rmin, ridx

    carry = (jnp.full((8, ROWS), 3.0e38, jnp.float32),
             jnp.zeros((8, ROWS), jnp.float32))
    for j in range(NCH):
        carry = chunk(jnp.int32(j), carry)
    rmin, ridx = carry

    idxf = ridx * 8.0 + subiota
    vals, idxs = rmin, idxf
    for half in (4, 2, 1):
        vt, vb = vals[:half, :], vals[half:2 * half, :]
        it, ib = idxs[:half, :], idxs[half:2 * half, :]
        take_b = jnp.logical_or(vb < vt,
                                jnp.logical_and(vb == vt, ib < it))
        vals = jnp.where(take_b, vb, vt)
        idxs = jnp.where(take_b, ib, it)

    idx_ref[0, 0, :] = idxs[0, :].astype(jnp.int32)

    @pl.when(i == 0)
    def _():
        minsum_ref[0, 0] = 0.0

    minsum_ref[0, 0] += jnp.sum(vals[0, :])

    @pl.when(i == GRID - 1)
    def _():
        minsum_ref[0, 0] = minsum_ref[0, 0] * (1.25 / (N_TOK * D))


def _dist_argmin(z_flat, embedding):
    return pl.pallas_call(
        _dist_argmin_body,
        grid=(GRID,),
        in_specs=[
            pl.BlockSpec((ROWS, D), lambda i: (i, 0)),
            pl.BlockSpec((K, D), lambda i: (0, 0)),
        ],
        out_specs=[
            pl.BlockSpec((1, 1, ROWS), lambda i: (i, 0, 0)),
            pl.BlockSpec(memory_space=pltpu.SMEM),
        ],
        out_shape=[
            jax.ShapeDtypeStruct((GRID, 1, ROWS), jnp.int32),
            jax.ShapeDtypeStruct((1, 1), jnp.float32),
        ],
        scratch_shapes=[
            pltpu.VMEM((K, 1), jnp.float32),
            pltpu.VMEM((K, D), jnp.float32),
        ],
    )(z_flat, embedding)


def _sc_gather(embedding, idx_grouped):
    mesh = plsc.VectorSubcoreMesh(core_axis_name="c", subcore_axis_name="s")

    @functools.partial(
        pl.kernel,
        mesh=mesh,
        out_type=jax.ShapeDtypeStruct((N_TOK, D), jnp.float32),
        scratch_types=[
            pltpu.VMEM((CH, ROWS_CH), jnp.int32),
            pltpu.VMEM((ROWS_CH, D), jnp.float32),
            pltpu.VMEM((ROWS_CH, D), jnp.float32),
            pltpu.SemaphoreType.DMA,
            pltpu.SemaphoreType.DMA,
        ],
    )
    def gather_k(table_hbm, idx_hbm, out_hbm, idx_v, rows_a, rows_b, gsem, ssem):
        wid = lax.axis_index("s") * 2 + lax.axis_index("c")
        base = wid * B_PER_W
        bufs = (rows_a, rows_b)
        pltpu.sync_copy(idx_hbm.at[wid], idx_v)
        g = pltpu.async_copy(table_hbm.at[idx_v.at[0]], bufs[0], gsem)
        prev_s = None
        for c in range(CH):
            if prev_s is not None:
                prev_s.wait()
            if c + 1 < CH:
                g_next = pltpu.async_copy(table_hbm.at[idx_v.at[c + 1]],
                                          bufs[(c + 1) % 2], gsem)
            g.wait()
            prev_s = pltpu.async_copy(
                bufs[c % 2], out_hbm.at[pl.ds(base + c * ROWS_CH, ROWS_CH)], ssem)
            if c + 1 < CH:
                g = g_next
        prev_s.wait()

    return gather_k(embedding, idx_grouped)


def kernel(z_flat, embedding):
    idx3, minsum = _dist_argmin(z_flat, embedding)
    indices = idx3.reshape(N_TOK)
    loss = minsum[0, 0]
    quantized = _sc_gather(embedding, indices.reshape(NW, CH, ROWS_CH))
    return (loss, quantized, indices)
